# baseline (speedup 1.0000x reference)
import functools

import jax
import jax.numpy as jnp
from jax import lax
from jax.experimental import pallas as pl
from jax.experimental.pallas import tpu as pltpu
from jax.experimental.pallas import tpu_sc as plsc

_LANES = 128
_ROWS_PER_BLOCK = 8192


def _sc_deint_build(BN):
    info = plsc.get_sparse_core_info()
    NC, NS, L = info.num_cores, info.num_subcores, info.num_lanes
    NW = NC * NS
    rows = BN // NW
    CH = 256
    nch = rows // CH
    SR = BN // _LANES
    mesh = plsc.VectorSubcoreMesh(core_axis_name="c", subcore_axis_name="s")

    @functools.partial(
        pl.kernel, mesh=mesh,
        compiler_params=pltpu.CompilerParams(needs_layout_passes=False),
        out_type=jax.ShapeDtypeStruct((8, SR, _LANES), jnp.float32),
        scratch_types=[
            pltpu.VMEM((CH, 4), jnp.float32),
            pltpu.VMEM((CH, 4), jnp.float32),
            pltpu.VMEM((8, 4 * CH // _LANES, _LANES), jnp.float32),
        ],
    )
    def deint(rp_hbm, rt_hbm, out_hbm, rp_v, rt_v, out_v):
        wid = lax.axis_index("s") * NC + lax.axis_index("c")
        base = wid * rows
        iota = lax.iota(jnp.int32, L)
        for ch2 in range(nch // 4):
            for h in range(4):
                off = base + (4 * ch2 + h) * CH
                pltpu.sync_copy(rp_hbm.at[pl.ds(off, CH)], rp_v)
                pltpu.sync_copy(rt_hbm.at[pl.ds(off, CH)], rt_v)
                for c in range(4):
                    colidx = jnp.full((L,), c, jnp.int32)
                    for r3 in range(CH // _LANES):
                        def body(k, carry, c=c, colidx=colidx, r3=r3, h=h):
                            rowidx = r3 * _LANES + k * L + iota
                            vp = plsc.load_gather(rp_v, [rowidx, colidx])
                            vt = plsc.load_gather(rt_v, [rowidx, colidx])
                            ro = h * (CH // _LANES) + r3
                            out_v[c, ro, pl.ds(k * L, L)] = vp
                            out_v[4 + c, ro, pl.ds(k * L, L)] = vt
                            return carry
                        lax.fori_loop(0, _LANES // L, body, 0)
            for c in range(8):
                pltpu.sync_copy(
                    out_v.at[c],
                    out_hbm.at[c, pl.ds(
                        pl.multiple_of((base + 4 * ch2 * CH) // _LANES, 8),
                        4 * CH // _LANES)])

    return deint


def _focal_body(x_ref, tg1_ref, out_ref):
    x = x_ref[...]
    tg1 = tg1_ref[...]
    cls_iota = jax.lax.broadcasted_iota(jnp.int32, x.shape, 1)
    m = cls_iota == (tg1 - 1)
    e = jnp.exp(-jnp.abs(x))
    u = 1.0 / (1.0 + e)
    v = e * u
    a = u * u
    b = v * v
    s = x >= 0
    w1 = jnp.where(s, a, b)
    w2 = (a + b) - w1
    lg = jnp.log1p(e)
    mx = jnp.maximum(x, 0.0)
    ce0 = mx + lg
    ce1 = (mx - x) + lg
    fsum = 0.25 * jnp.sum(jnp.where(m, ce1 * w2, 3.0 * (ce0 * w1)))
    out_ref[0, 0, 0] = fsum


def _boxes_body(fp_ref, tg2_ref, rr_ref, cn_ref, out_ref):
    tg2 = tg2_ref[...]
    posf = (tg2 > 0).astype(jnp.float32)
    npos = jnp.sum(posf)

    p0 = rr_ref[0]; p1 = rr_ref[1]; p2 = rr_ref[2]; p3 = rr_ref[3]
    t0 = rr_ref[4]; t1 = rr_ref[5]; t2 = rr_ref[6]; t3 = rr_ref[7]
    lr_min = jnp.minimum(t0, t2); lr_max = jnp.maximum(t0, t2)
    tb_min = jnp.minimum(t1, t3); tb_max = jnp.maximum(t1, t3)
    cness_t = jnp.sqrt(lr_min / lr_max * (tb_min / tb_max))

    x1 = -p0; y1 = -p1; x2 = p2; y2 = p3
    x1g = -t0; y1g = -t1; x2g = t2; y2g = t3
    xi1 = jnp.maximum(x1, x1g); yi1 = jnp.maximum(y1, y1g)
    xi2 = jnp.minimum(x2, x2g); yi2 = jnp.minimum(y2, y2g)
    inter = jnp.where((yi2 > yi1) & (xi2 > xi1), (xi2 - xi1) * (yi2 - yi1), 0.0)
    union = (x2 - x1) * (y2 - y1) + (x2g - x1g) * (y2g - y1g) - inter
    iou = inter / (union + 1e-7)
    xc1 = jnp.minimum(x1, x1g); yc1 = jnp.minimum(y1, y1g)
    xc2 = jnp.maximum(x2, x2g); yc2 = jnp.maximum(y2, y2g)
    diag = (xc2 - xc1) ** 2 + (yc2 - yc1) ** 2 + 1e-7
    cdist = ((x1 + x2) / 2.0 - (x1g + x2g) / 2.0) ** 2 + \
            ((y1 + y2) / 2.0 - (y1g + y2g) / 2.0) ** 2
    diou = 1.0 - iou + cdist / diag
    w = cness_t * posf
    rnum = jnp.sum(diou * w)
    rden = jnp.sum(w)

    cn = cn_ref[...]
    bce = jnp.maximum(cn, 0.0) - cn * cness_t + jnp.log1p(jnp.exp(-jnp.abs(cn)))
    csum = jnp.sum(bce * posf)

    nblk = fp_ref.shape[0]

    def acc(i, t):
        return t + fp_ref[i, 0, 0]

    fsum = lax.fori_loop(0, nblk, acc, 0.0)

    denom = jnp.maximum(npos, 1.0)
    cls_loss = fsum / denom
    reg_loss = rnum / (rden + 1e-8)
    cness_loss = csum / denom
    out_ref[0] = cls_loss
    out_ref[1] = reg_loss
    out_ref[2] = cness_loss
    out_ref[3] = cls_loss + reg_loss + cness_loss
    out_ref[4] = 0.0
    out_ref[5] = 0.0
    out_ref[6] = 0.0
    out_ref[7] = 0.0


def kernel(cls_logits, reg_preds, cness_preds, cls_tgts, reg_tgts):
    B, N, C = cls_logits.shape
    BN = B * N
    R = _ROWS_PER_BLOCK
    assert BN % R == 0 and BN % _LANES == 0
    grid = BN // R
    SR = BN // _LANES

    x = cls_logits.reshape(BN, C)
    tg1 = cls_tgts.reshape(BN, 1).astype(jnp.int32)
    tg2 = cls_tgts.reshape(SR, _LANES).astype(jnp.int32)
    rr = _sc_deint_build(BN)(reg_preds.reshape(BN, 4), reg_tgts.reshape(BN, 4))
    cn = cness_preds.reshape(SR, _LANES)

    fpart = pl.pallas_call(
        _focal_body,
        grid=(grid,),
        in_specs=[
            pl.BlockSpec((R, C), lambda i: (i, 0)),
            pl.BlockSpec((R, 1), lambda i: (i, 0)),
        ],
        out_specs=pl.BlockSpec((1, 1, 8), lambda i: (i, 0, 0), memory_space=pltpu.SMEM),
        out_shape=jax.ShapeDtypeStruct((grid, 1, 8), jnp.float32),
        compiler_params=pltpu.CompilerParams(
            dimension_semantics=("arbitrary",),
        ),
        interpret=False,
    )(x, tg1)

    out = pl.pallas_call(
        _boxes_body,
        in_specs=[
            pl.BlockSpec(memory_space=pltpu.SMEM),
            pl.BlockSpec((SR, _LANES), lambda: (0, 0)),
            pl.BlockSpec((8, SR, _LANES), lambda: (0, 0, 0)),
            pl.BlockSpec((SR, _LANES), lambda: (0, 0)),
        ],
        out_specs=pl.BlockSpec(memory_space=pltpu.SMEM),
        out_shape=jax.ShapeDtypeStruct((8,), jnp.float32),
        interpret=False,
    )(fpart, tg2, rr, cn)

    return out[0], out[1], out[2], out[3]


# speedup vs baseline: 1.4226x; 1.4226x over previous
import jax
import jax.numpy as jnp
from jax import lax
from jax.experimental import pallas as pl
from jax.experimental.pallas import tpu as pltpu

_LANES = 128
_ROWS_PER_BLOCK = 8192


def _focal_body(x_ref, tg1_ref, out_ref):
    x = x_ref[...]
    tg1 = tg1_ref[...]
    cls_iota = jax.lax.broadcasted_iota(jnp.int32, x.shape, 1)
    m = cls_iota == (tg1 - 1)
    e = jnp.exp(-jnp.abs(x))
    u = 1.0 / (1.0 + e)
    v = e * u
    a = u * u
    b = v * v
    s = x >= 0
    w1 = jnp.where(s, a, b)
    w2 = (a + b) - w1
    lg = jnp.log1p(e)
    mx = jnp.maximum(x, 0.0)
    ce0 = mx + lg
    ce1 = (mx - x) + lg
    fsum = 0.25 * jnp.sum(jnp.where(m, ce1 * w2, 3.0 * (ce0 * w1)))
    out_ref[0, 0, 0] = fsum


def _boxes_body(fp_ref, tg2_ref, rr_ref, cn_ref, out_ref):
    tg2 = tg2_ref[...]
    posf = (tg2 > 0).astype(jnp.float32)
    npos = jnp.sum(posf)

    p0 = rr_ref[0]; p1 = rr_ref[1]; p2 = rr_ref[2]; p3 = rr_ref[3]
    t0 = rr_ref[4]; t1 = rr_ref[5]; t2 = rr_ref[6]; t3 = rr_ref[7]
    lr_min = jnp.minimum(t0, t2); lr_max = jnp.maximum(t0, t2)
    tb_min = jnp.minimum(t1, t3); tb_max = jnp.maximum(t1, t3)
    cness_t = jnp.sqrt(lr_min / lr_max * (tb_min / tb_max))

    x1 = -p0; y1 = -p1; x2 = p2; y2 = p3
    x1g = -t0; y1g = -t1; x2g = t2; y2g = t3
    xi1 = jnp.maximum(x1, x1g); yi1 = jnp.maximum(y1, y1g)
    xi2 = jnp.minimum(x2, x2g); yi2 = jnp.minimum(y2, y2g)
    inter = jnp.where((yi2 > yi1) & (xi2 > xi1), (xi2 - xi1) * (yi2 - yi1), 0.0)
    union = (x2 - x1) * (y2 - y1) + (x2g - x1g) * (y2g - y1g) - inter
    iou = inter / (union + 1e-7)
    xc1 = jnp.minimum(x1, x1g); yc1 = jnp.minimum(y1, y1g)
    xc2 = jnp.maximum(x2, x2g); yc2 = jnp.maximum(y2, y2g)
    diag = (xc2 - xc1) ** 2 + (yc2 - yc1) ** 2 + 1e-7
    cdist = ((x1 + x2) / 2.0 - (x1g + x2g) / 2.0) ** 2 + \
            ((y1 + y2) / 2.0 - (y1g + y2g) / 2.0) ** 2
    diou = 1.0 - iou + cdist / diag
    w = cness_t * posf
    rnum = jnp.sum(diou * w)
    rden = jnp.sum(w)

    cn = cn_ref[...]
    bce = jnp.maximum(cn, 0.0) - cn * cness_t + jnp.log1p(jnp.exp(-jnp.abs(cn)))
    csum = jnp.sum(bce * posf)

    nblk = fp_ref.shape[0]

    def acc(i, t):
        return t + fp_ref[i, 0, 0]

    fsum = lax.fori_loop(0, nblk, acc, 0.0)

    denom = jnp.maximum(npos, 1.0)
    cls_loss = fsum / denom
    reg_loss = rnum / (rden + 1e-8)
    cness_loss = csum / denom
    out_ref[0] = cls_loss
    out_ref[1] = reg_loss
    out_ref[2] = cness_loss
    out_ref[3] = cls_loss + reg_loss + cness_loss
    out_ref[4] = 0.0
    out_ref[5] = 0.0
    out_ref[6] = 0.0
    out_ref[7] = 0.0


def kernel(cls_logits, reg_preds, cness_preds, cls_tgts, reg_tgts):
    B, N, C = cls_logits.shape
    BN = B * N
    R = _ROWS_PER_BLOCK
    assert BN % R == 0 and BN % _LANES == 0
    grid = BN // R
    SR = BN // _LANES

    x = cls_logits.reshape(BN, C)
    tg1 = cls_tgts.reshape(BN, 1).astype(jnp.int32)
    tg2 = cls_tgts.reshape(SR, _LANES).astype(jnp.int32)
    rr = jnp.concatenate(
        [reg_preds.reshape(BN, 4).T, reg_tgts.reshape(BN, 4).T], axis=0
    ).reshape(8, SR, _LANES)
    cn = cness_preds.reshape(SR, _LANES)

    fpart = pl.pallas_call(
        _focal_body,
        grid=(grid,),
        in_specs=[
            pl.BlockSpec((R, C), lambda i: (i, 0)),
            pl.BlockSpec((R, 1), lambda i: (i, 0)),
        ],
        out_specs=pl.BlockSpec((1, 1, 8), lambda i: (i, 0, 0), memory_space=pltpu.SMEM),
        out_shape=jax.ShapeDtypeStruct((grid, 1, 8), jnp.float32),
        compiler_params=pltpu.CompilerParams(
            dimension_semantics=("arbitrary",),
        ),
        interpret=False,
    )(x, tg1)

    out = pl.pallas_call(
        _boxes_body,
        in_specs=[
            pl.BlockSpec(memory_space=pltpu.SMEM),
            pl.BlockSpec((SR, _LANES), lambda: (0, 0)),
            pl.BlockSpec((8, SR, _LANES), lambda: (0, 0, 0)),
            pl.BlockSpec((SR, _LANES), lambda: (0, 0)),
        ],
        out_specs=pl.BlockSpec(memory_space=pltpu.SMEM),
        out_shape=jax.ShapeDtypeStruct((8,), jnp.float32),
        interpret=False,
    )(fpart, tg2, rr, cn)

    return out[0], out[1], out[2], out[3]
